# Initial kernel scaffold; baseline (speedup 1.0000x reference)
import jax
import jax.numpy as jnp
from jax.experimental import pallas as pl


def kernel(x, edge_index, Wl, bl, Wr, Wa, ba, W1, b1, W2, b2, W3, b3):
    raise NotImplementedError("write your pallas kernel here")


# speedup vs baseline: 4.3098x; 4.3098x over previous
import functools

import jax
import jax.numpy as jnp
from jax import lax
from jax.experimental import pallas as pl
from jax.experimental.pallas import tpu as pltpu
from jax.experimental.pallas import tpu_sc as plsc

N = 10000
E = 160000
NTILES = 16
NPT = 640
NPAD = NTILES * NPT
CROWS = NPAD // 128
CHUNK = 64
GROUP = 8
NGROUP = 20
EPT_PAD = GROUP * NGROUP * CHUNK
E_PAD = EPT_PAD * NTILES


def _project(x, wcat):
    rb = 1024

    def body(x_ref, w_ref, yl_ref, yr_ref, r_ref):
        y = jnp.dot(x_ref[...], w_ref[...], preferred_element_type=jnp.float32)
        yl_ref[...] = y[:, :128]
        yr_ref[...] = y[:, 128:256]
        r_ref[...] = y[:, 256:]

    return pl.pallas_call(
        body,
        grid=(pl.cdiv(N, rb),),
        in_specs=[
            pl.BlockSpec((rb, 512), lambda i: (i, 0)),
            pl.BlockSpec((512, 512), lambda i: (0, 0)),
        ],
        out_specs=[
            pl.BlockSpec((rb, 128), lambda i: (i, 0)),
            pl.BlockSpec((rb, 128), lambda i: (i, 0)),
            pl.BlockSpec((rb, 256), lambda i: (i, 0)),
        ],
        out_shape=[
            jax.ShapeDtypeStruct((N, 128), jnp.float32),
            jax.ShapeDtypeStruct((N, 128), jnp.float32),
            jax.ShapeDtypeStruct((N, 256), jnp.float32),
        ],
    )(x, wcat)


def _sc_aggregate(yl, yr, srcs, dsts, zrow):
    mesh = plsc.VectorSubcoreMesh(core_axis_name="c", subcore_axis_name="s")

    @functools.partial(
        pl.kernel,
        out_type=(
            jax.ShapeDtypeStruct((NPAD, 128), jnp.float32),
            jax.ShapeDtypeStruct((NPAD, 128), jnp.float32),
        ),
        mesh=mesh,
        scratch_types=[
            pltpu.VMEM_SHARED((NPAD, 128), jnp.float32),
            pltpu.VMEM((GROUP, CHUNK), jnp.int32),
            pltpu.VMEM((GROUP, CHUNK), jnp.int32),
            pltpu.VMEM((CHUNK, 128), jnp.float32),
            pltpu.SemaphoreType.DMA,
        ],
    )
    def k(yl_h, yr_h, srcs_h, dsts_h, zrow_h, aggl_h, aggr_h,
          agg_s, src_v, dst_v, rowbuf, sem):
        c = lax.axis_index("c")
        s = lax.axis_index("s")
        base = s * NPT
        sl = pl.ds(base, NPT)
        pltpu.sync_copy(zrow_h, agg_s.at[sl])
        plsc.subcore_barrier()

        def run(y_h):
            def group(g, carry):
                pltpu.sync_copy(srcs_h.at[s * NGROUP + g], src_v)
                pltpu.sync_copy(dsts_h.at[s * NGROUP + g], dst_v)
                for j in range(GROUP):
                    pltpu.async_copy(y_h.at[src_v.at[j]], rowbuf, sem).wait()
                    pltpu.sync_copy(rowbuf, agg_s.at[dst_v.at[j]], add=True)
                return carry
            lax.fori_loop(0, NGROUP, group, 0)

        @pl.when(c == 0)
        def _():
            run(yl_h)

        @pl.when(c == 1)
        def _():
            run(yr_h)

        plsc.subcore_barrier()

        @pl.when(c == 0)
        def _():
            pltpu.sync_copy(agg_s.at[sl], aggl_h.at[sl])

        @pl.when(c == 1)
        def _():
            pltpu.sync_copy(agg_s.at[sl], aggr_h.at[sl])

    return k(yl, yr, srcs, dsts, zrow)


def _sc_count(dsts, zrow, ones):
    mesh = plsc.VectorSubcoreMesh(core_axis_name="c", subcore_axis_name="s")
    gpw = NTILES * NGROUP // 32

    @functools.partial(
        pl.kernel,
        out_type=(
            jax.ShapeDtypeStruct((NPAD, 128), jnp.float32),
            jax.ShapeDtypeStruct((NPAD, 128), jnp.float32),
        ),
        mesh=mesh,
        scratch_types=[
            pltpu.VMEM_SHARED((NPAD, 128), jnp.float32),
            pltpu.VMEM((GROUP, CHUNK), jnp.int32),
            pltpu.VMEM((CHUNK, 128), jnp.float32),
        ],
    )
    def k(dsts_h, zrow_h, ones_h, cntl_h, cntr_h, cnt_s, dst_v, ones_v):
        c = lax.axis_index("c")
        s = lax.axis_index("s")
        base = s * NPT
        sl = pl.ds(base, NPT)
        gbase = (c * NTILES + s) * gpw
        pltpu.sync_copy(zrow_h, cnt_s.at[sl])
        pltpu.sync_copy(ones_h, ones_v)
        plsc.subcore_barrier()

        def group(g, carry):
            pltpu.sync_copy(dsts_h.at[gbase + g], dst_v)
            for j in range(GROUP):
                pltpu.sync_copy(ones_v, cnt_s.at[dst_v.at[j]], add=True)
            return carry
        lax.fori_loop(0, gpw, group, 0)

        plsc.subcore_barrier()

        @pl.when(c == 0)
        def _():
            pltpu.sync_copy(cnt_s.at[sl], cntl_h.at[sl])

        @pl.when(c == 1)
        def _():
            pltpu.sync_copy(cnt_s.at[sl], cntr_h.at[sl])

    return k(dsts, zrow, ones)


def _mlp_factors(aggl, aggr, cntl, cntr, r, bl2, wat, ba2, w1t, b12, w2t,
                 b22, w3t, b32):
    rb = 1024

    def body(al, ar, cl, cr, rr, bl_r, wa_r, ba_r, w1_r, b1_r, w2_r, b2_r,
             w3_r, b3_r, p_ref, qt_ref):
        cntc = jnp.maximum(cl[:, :1] + cr[:, :1], 1.0)
        agg = jnp.concatenate([al[...], ar[...]], axis=1) / cntc
        h = jnp.maximum(agg + bl_r[...] + rr[...], 0.0)
        h = jnp.maximum(
            jnp.dot(h, wa_r[...], preferred_element_type=jnp.float32)
            + ba_r[...], 0.0)
        h = jnp.maximum(
            jnp.dot(h, w1_r[...], preferred_element_type=jnp.float32)
            + b1_r[...], 0.0)
        h = jnp.maximum(
            jnp.dot(h, w2_r[...], preferred_element_type=jnp.float32)
            + b2_r[...], 0.0)
        z = (jnp.dot(h, w3_r[...], preferred_element_type=jnp.float32)
             + b3_r[...])
        sq = jnp.sum(z * z, axis=1, keepdims=True)
        col = lax.broadcasted_iota(jnp.int32, z.shape, 1)
        p = jnp.where(col < 3, -2.0 * z,
                      jnp.where(col == 3, sq,
                                jnp.where(col == 4, 1.0, 0.0)))
        q = jnp.where(col < 3, z,
                      jnp.where(col == 3, 1.0,
                                jnp.where(col == 4, sq, 0.0)))
        p_ref[...] = p
        qt_ref[...] = q.T

    full = lambda shape: pl.BlockSpec(shape, lambda i: (0, 0))
    return pl.pallas_call(
        body,
        grid=(pl.cdiv(N, rb),),
        in_specs=[
            pl.BlockSpec((rb, 128), lambda i: (i, 0)),
            pl.BlockSpec((rb, 128), lambda i: (i, 0)),
            pl.BlockSpec((rb, 128), lambda i: (i, 0)),
            pl.BlockSpec((rb, 128), lambda i: (i, 0)),
            pl.BlockSpec((rb, 256), lambda i: (i, 0)),
            full((1, 256)), full((256, 128)), full((1, 128)),
            full((128, 64)), full((1, 64)), full((64, 32)), full((1, 32)),
            full((32, 8)), full((1, 8)),
        ],
        out_specs=[
            pl.BlockSpec((rb, 8), lambda i: (i, 0)),
            pl.BlockSpec((8, rb), lambda i: (0, i)),
        ],
        out_shape=[
            jax.ShapeDtypeStruct((N, 8), jnp.float32),
            jax.ShapeDtypeStruct((8, N), jnp.float32),
        ],
    )(aggl, aggr, cntl, cntr, r, bl2, wat, ba2, w1t, b12, w2t, b22, w3t,
      b32)


def _cdist(p, qt):
    rb, cb = 512, 2048

    def body(p_ref, qt_ref, o_ref):
        d2 = jnp.dot(p_ref[...], qt_ref[...],
                     preferred_element_type=jnp.float32)
        o_ref[...] = jnp.sqrt(jnp.maximum(d2, 1e-24))

    return pl.pallas_call(
        body,
        grid=(pl.cdiv(N, rb), pl.cdiv(N, cb)),
        in_specs=[
            pl.BlockSpec((rb, 8), lambda i, j: (i, 0)),
            pl.BlockSpec((8, cb), lambda i, j: (0, j)),
        ],
        out_specs=pl.BlockSpec((rb, cb), lambda i, j: (i, j)),
        out_shape=jax.ShapeDtypeStruct((N, N), jnp.float32),
    )(p, qt)


def kernel(x, edge_index, Wl, bl, Wr, Wa, ba, W1, b1, W2, b2, W3, b3):
    f32 = jnp.float32
    wcat = jnp.concatenate([Wl.T, Wr.T], axis=1)
    src = edge_index[0].astype(jnp.int32)
    dst = edge_index[1].astype(jnp.int32)
    npad_e = E_PAD - E
    pad_iota = jnp.arange(npad_e, dtype=jnp.int32)
    src_full = jnp.concatenate([src, pad_iota % N])
    dst_full = jnp.concatenate([dst, N + pad_iota % (NPAD - N)])
    srcs = src_full.reshape(NTILES * NGROUP, GROUP, CHUNK)
    dsts = dst_full.reshape(NTILES * NGROUP, GROUP, CHUNK)
    zrow = jnp.zeros((NPT, 128), f32)
    ones = jnp.ones((CHUNK, 128), f32)
    bl2 = bl.reshape(1, 256)
    ba2 = ba.reshape(1, 128)
    b12 = b1.reshape(1, 64)
    b22 = b2.reshape(1, 32)
    w3t = jnp.pad(W3.T, ((0, 0), (0, 5)))
    b32 = jnp.pad(b3, (0, 5)).reshape(1, 8)

    yl, yr, r = _project(x, wcat)
    aggl, aggr = _sc_aggregate(yl, yr, srcs, dsts, zrow)
    cntl, cntr = _sc_count(dsts, zrow, ones)
    pmat, qtmat = _mlp_factors(aggl, aggr, cntl, cntr, r, bl2, Wa.T, ba2,
                               W1.T, b12, W2.T, b22, w3t, b32)
    return _cdist(pmat, qtmat)


# speedup vs baseline: 4.9018x; 1.1374x over previous
import functools

import jax
import jax.numpy as jnp
from jax import lax
from jax.experimental import pallas as pl
from jax.experimental.pallas import tpu as pltpu
from jax.experimental.pallas import tpu_sc as plsc

N = 10000
E = 160000
NTILES = 16
NPT = 640
NPAD = NTILES * NPT
CROWS = NPAD // 128
CHUNK = 64
GROUP = 16
NGROUP = 10
EPT_PAD = GROUP * NGROUP * CHUNK
E_PAD = EPT_PAD * NTILES


def _project(x, wcat):
    rb = 1024

    def body(x_ref, w_ref, yl_ref, yr_ref, r_ref):
        y = jnp.dot(x_ref[...], w_ref[...], preferred_element_type=jnp.float32)
        yl_ref[...] = y[:, :128]
        yr_ref[...] = y[:, 128:256]
        r_ref[...] = y[:, 256:]

    return pl.pallas_call(
        body,
        grid=(pl.cdiv(N, rb),),
        in_specs=[
            pl.BlockSpec((rb, 512), lambda i: (i, 0)),
            pl.BlockSpec((512, 512), lambda i: (0, 0)),
        ],
        out_specs=[
            pl.BlockSpec((rb, 128), lambda i: (i, 0)),
            pl.BlockSpec((rb, 128), lambda i: (i, 0)),
            pl.BlockSpec((rb, 256), lambda i: (i, 0)),
        ],
        out_shape=[
            jax.ShapeDtypeStruct((N, 128), jnp.float32),
            jax.ShapeDtypeStruct((N, 128), jnp.float32),
            jax.ShapeDtypeStruct((N, 256), jnp.float32),
        ],
    )(x, wcat)


def _sc_aggregate(yl, yr, srcs, dsts, zrow):
    mesh = plsc.VectorSubcoreMesh(core_axis_name="c", subcore_axis_name="s")

    @functools.partial(
        pl.kernel,
        out_type=(
            jax.ShapeDtypeStruct((NPAD, 128), jnp.float32),
            jax.ShapeDtypeStruct((NPAD, 128), jnp.float32),
        ),
        mesh=mesh,
        scratch_types=[
            pltpu.VMEM_SHARED((NPAD, 128), jnp.float32),
            pltpu.VMEM((GROUP, CHUNK), jnp.int32),
            pltpu.VMEM((GROUP, CHUNK), jnp.int32),
            pltpu.VMEM((CHUNK, 128), jnp.float32),
            pltpu.VMEM((CHUNK, 128), jnp.float32),
            pltpu.SemaphoreType.DMA,
            pltpu.SemaphoreType.DMA,
            pltpu.SemaphoreType.DMA,
            pltpu.SemaphoreType.DMA,
        ],
    )
    def k(yl_h, yr_h, srcs_h, dsts_h, zrow_h, aggl_h, aggr_h,
          agg_s, src_v, dst_v, rowbuf0, rowbuf1, gsem0, gsem1, ssem0,
          ssem1):
        c = lax.axis_index("c")
        s = lax.axis_index("s")
        base = s * NPT
        sl = pl.ds(base, NPT)
        pltpu.sync_copy(zrow_h, agg_s.at[sl])
        plsc.subcore_barrier()

        bufs = (rowbuf0, rowbuf1)
        gsems = (gsem0, gsem1)
        ssems = (ssem0, ssem1)

        def run(y_h):
            def group(g, carry):
                pltpu.sync_copy(srcs_h.at[s * NGROUP + g], src_v)
                pltpu.sync_copy(dsts_h.at[s * NGROUP + g], dst_v)
                gh = [None] * GROUP
                sh = [None] * GROUP
                gh[0] = pltpu.async_copy(y_h.at[src_v.at[0]], bufs[0],
                                         gsems[0])
                for j in range(GROUP):
                    b = j % 2
                    gh[j].wait()
                    if j + 1 < GROUP:
                        if j >= 1:
                            sh[j - 1].wait()
                        gh[j + 1] = pltpu.async_copy(
                            y_h.at[src_v.at[j + 1]], bufs[1 - b],
                            gsems[1 - b])
                    sh[j] = pltpu.async_copy(bufs[b],
                                             agg_s.at[dst_v.at[j]],
                                             ssems[b], add=True)
                sh[GROUP - 2].wait()
                sh[GROUP - 1].wait()
                return carry
            lax.fori_loop(0, NGROUP, group, 0)

        @pl.when(c == 0)
        def _():
            run(yl_h)

        @pl.when(c == 1)
        def _():
            run(yr_h)

        plsc.subcore_barrier()

        @pl.when(c == 0)
        def _():
            pltpu.sync_copy(agg_s.at[sl], aggl_h.at[sl])

        @pl.when(c == 1)
        def _():
            pltpu.sync_copy(agg_s.at[sl], aggr_h.at[sl])

    return k(yl, yr, srcs, dsts, zrow)


def _sc_count(dsts, zrow, ones):
    mesh = plsc.VectorSubcoreMesh(core_axis_name="c", subcore_axis_name="s")
    gpw = NTILES * NGROUP // 32

    @functools.partial(
        pl.kernel,
        out_type=(
            jax.ShapeDtypeStruct((NPAD, 128), jnp.float32),
            jax.ShapeDtypeStruct((NPAD, 128), jnp.float32),
        ),
        mesh=mesh,
        scratch_types=[
            pltpu.VMEM_SHARED((NPAD, 128), jnp.float32),
            pltpu.VMEM((GROUP, CHUNK), jnp.int32),
            pltpu.VMEM((CHUNK, 128), jnp.float32),
            pltpu.SemaphoreType.DMA,
        ],
    )
    def k(dsts_h, zrow_h, ones_h, cntl_h, cntr_h, cnt_s, dst_v, ones_v,
          sem):
        c = lax.axis_index("c")
        s = lax.axis_index("s")
        base = s * NPT
        sl = pl.ds(base, NPT)
        gbase = (c * NTILES + s) * gpw
        pltpu.sync_copy(zrow_h, cnt_s.at[sl])
        pltpu.sync_copy(ones_h, ones_v)
        plsc.subcore_barrier()

        def group(g, carry):
            pltpu.sync_copy(dsts_h.at[gbase + g], dst_v)
            hs = [pltpu.async_copy(ones_v, cnt_s.at[dst_v.at[j]], sem,
                                   add=True)
                  for j in range(GROUP)]
            for h in hs:
                h.wait()
            return carry
        lax.fori_loop(0, gpw, group, 0)

        plsc.subcore_barrier()

        @pl.when(c == 0)
        def _():
            pltpu.sync_copy(cnt_s.at[sl], cntl_h.at[sl])

        @pl.when(c == 1)
        def _():
            pltpu.sync_copy(cnt_s.at[sl], cntr_h.at[sl])

    return k(dsts, zrow, ones)


def _mlp_factors(aggl, aggr, cntl, cntr, r, bl2, wat, ba2, w1t, b12, w2t,
                 b22, w3t, b32):
    rb = 1024

    def body(al, ar, cl, cr, rr, bl_r, wa_r, ba_r, w1_r, b1_r, w2_r, b2_r,
             w3_r, b3_r, p_ref, qt_ref):
        cntc = jnp.maximum(cl[:, :1] + cr[:, :1], 1.0)
        agg = jnp.concatenate([al[...], ar[...]], axis=1) / cntc
        h = jnp.maximum(agg + bl_r[...] + rr[...], 0.0)
        h = jnp.maximum(
            jnp.dot(h, wa_r[...], preferred_element_type=jnp.float32)
            + ba_r[...], 0.0)
        h = jnp.maximum(
            jnp.dot(h, w1_r[...], preferred_element_type=jnp.float32)
            + b1_r[...], 0.0)
        h = jnp.maximum(
            jnp.dot(h, w2_r[...], preferred_element_type=jnp.float32)
            + b2_r[...], 0.0)
        z = (jnp.dot(h, w3_r[...], preferred_element_type=jnp.float32)
             + b3_r[...])
        sq = jnp.sum(z * z, axis=1, keepdims=True)
        col = lax.broadcasted_iota(jnp.int32, z.shape, 1)
        p = jnp.where(col < 3, -2.0 * z,
                      jnp.where(col == 3, sq,
                                jnp.where(col == 4, 1.0, 0.0)))
        q = jnp.where(col < 3, z,
                      jnp.where(col == 3, 1.0,
                                jnp.where(col == 4, sq, 0.0)))
        p_ref[...] = p
        qt_ref[...] = q.T

    full = lambda shape: pl.BlockSpec(shape, lambda i: (0, 0))
    return pl.pallas_call(
        body,
        grid=(pl.cdiv(N, rb),),
        in_specs=[
            pl.BlockSpec((rb, 128), lambda i: (i, 0)),
            pl.BlockSpec((rb, 128), lambda i: (i, 0)),
            pl.BlockSpec((rb, 128), lambda i: (i, 0)),
            pl.BlockSpec((rb, 128), lambda i: (i, 0)),
            pl.BlockSpec((rb, 256), lambda i: (i, 0)),
            full((1, 256)), full((256, 128)), full((1, 128)),
            full((128, 64)), full((1, 64)), full((64, 32)), full((1, 32)),
            full((32, 8)), full((1, 8)),
        ],
        out_specs=[
            pl.BlockSpec((rb, 8), lambda i: (i, 0)),
            pl.BlockSpec((8, rb), lambda i: (0, i)),
        ],
        out_shape=[
            jax.ShapeDtypeStruct((N, 8), jnp.float32),
            jax.ShapeDtypeStruct((8, N), jnp.float32),
        ],
    )(aggl, aggr, cntl, cntr, r, bl2, wat, ba2, w1t, b12, w2t, b22, w3t,
      b32)


def _cdist(p, qt):
    rb, cb = 512, 2048

    def body(p_ref, qt_ref, o_ref):
        d2 = jnp.dot(p_ref[...], qt_ref[...],
                     preferred_element_type=jnp.float32)
        o_ref[...] = jnp.sqrt(jnp.maximum(d2, 1e-24))

    return pl.pallas_call(
        body,
        grid=(pl.cdiv(N, rb), pl.cdiv(N, cb)),
        in_specs=[
            pl.BlockSpec((rb, 8), lambda i, j: (i, 0)),
            pl.BlockSpec((8, cb), lambda i, j: (0, j)),
        ],
        out_specs=pl.BlockSpec((rb, cb), lambda i, j: (i, j)),
        out_shape=jax.ShapeDtypeStruct((N, N), jnp.float32),
    )(p, qt)


def kernel(x, edge_index, Wl, bl, Wr, Wa, ba, W1, b1, W2, b2, W3, b3):
    f32 = jnp.float32
    wcat = jnp.concatenate([Wl.T, Wr.T], axis=1)
    src = edge_index[0].astype(jnp.int32)
    dst = edge_index[1].astype(jnp.int32)
    npad_e = E_PAD - E
    pad_iota = jnp.arange(npad_e, dtype=jnp.int32)
    src_full = jnp.concatenate([src, pad_iota % N])
    dst_full = jnp.concatenate([dst, N + pad_iota % (NPAD - N)])
    srcs = src_full.reshape(NTILES * NGROUP, GROUP, CHUNK)
    dsts = dst_full.reshape(NTILES * NGROUP, GROUP, CHUNK)
    zrow = jnp.zeros((NPT, 128), f32)
    ones = jnp.ones((CHUNK, 128), f32)
    bl2 = bl.reshape(1, 256)
    ba2 = ba.reshape(1, 128)
    b12 = b1.reshape(1, 64)
    b22 = b2.reshape(1, 32)
    w3t = jnp.pad(W3.T, ((0, 0), (0, 5)))
    b32 = jnp.pad(b3, (0, 5)).reshape(1, 8)

    yl, yr, r = _project(x, wcat)
    aggl, aggr = _sc_aggregate(yl, yr, srcs, dsts, zrow)
    cntl, cntr = _sc_count(dsts, zrow, ones)
    pmat, qtmat = _mlp_factors(aggl, aggr, cntl, cntr, r, bl2, Wa.T, ba2,
                               W1.T, b12, W2.T, b22, w3t, b32)
    return _cdist(pmat, qtmat)
